# baseline (speedup 1.0000x reference)
import jax
import jax.numpy as jnp
from jax.experimental import pallas as pl
from jax.experimental.pallas import tpu as pltpu

_VOCAB = 128
_W = 16


def _fofe_block(ff_ref, x_ref, o_ref):
    ch = x_ref[...]
    R = ch.shape[0]
    ff = ff_ref[0]
    iota = jax.lax.broadcasted_iota(jnp.int32, (R, _VOCAB), 1)
    acc = jnp.zeros((R, _VOCAB), jnp.float32)
    p = jnp.ones((R, 1), jnp.float32)
    for w in range(_W - 1, -1, -1):
        chw = ch[:, w : w + 1]
        mw = chw != 0
        wgt = jnp.where(mw, p, jnp.float32(0.0))
        acc = acc + jnp.where(chw == iota, wgt, 0.0)
        p = p * jnp.where(mw, ff, jnp.float32(1.0))
    o_ref[...] = acc


def kernel(sents, lengths, forgetting_factor):
    B, S, Wd = sents.shape
    N = B * S
    R = 512
    x = sents.reshape(N, Wd).astype(jnp.int32)
    ff = forgetting_factor.astype(jnp.float32)
    out = pl.pallas_call(
        _fofe_block,
        grid=(N // R,),
        in_specs=[
            pl.BlockSpec(memory_space=pltpu.SMEM),
            pl.BlockSpec((R, Wd), lambda i: (i, 0)),
        ],
        out_specs=pl.BlockSpec((R, _VOCAB), lambda i: (i, 0)),
        out_shape=jax.ShapeDtypeStruct((N, _VOCAB), jnp.float32),
    )(ff, x)
    return out.reshape(B, S, _VOCAB), lengths


# speedup vs baseline: 4.1427x; 4.1427x over previous
import functools

import jax
import jax.numpy as jnp
from jax import lax
from jax.experimental import pallas as pl
from jax.experimental.pallas import tpu as pltpu
from jax.experimental.pallas import tpu_sc as plsc

_VOCAB = 128
_W = 16
_L = 16
_NC = 2
_NS = 16
_NWORKERS = _NC * _NS
_N_WORDS = 16 * 2048
_WPW = _N_WORDS // _NWORKERS
_CHUNK = 256
_NCHUNK = _WPW // _CHUNK
_GROUPS = _CHUNK // _L


def _fofe_body(ff_hbm, x_hbm, out_hbm, ff_v, chars_v, outbuf):
    cid = lax.axis_index("c")
    sid = lax.axis_index("s")
    wid = sid * _NC + cid
    base_word = wid * _WPW

    pltpu.sync_copy(ff_hbm, ff_v)
    pltpu.sync_copy(x_hbm.at[pl.ds(base_word * _W, _WPW * _W)], chars_v)

    ffv = ff_v[...]
    ones = jnp.ones((_L,), jnp.float32)
    zeros = jnp.zeros((_L,), jnp.float32)
    lane = lax.iota(jnp.int32, _L)
    lane_w = lane * _W
    lane_row = lane * _VOCAB

    def zero_body(i, carry):
        for j in range(16):
            outbuf[pl.ds(i * 256 + j * 16, 16)] = zeros
        return carry

    lax.fori_loop(0, _CHUNK * _VOCAB // 256, zero_body, 0)

    for chunk in range(_NCHUNK):
        cbase = chunk * _CHUNK * _W

        def fill_body(g, carry):
            cidx0 = cbase + g * (_L * _W) + lane_w
            idx0 = g * (_L * _VOCAB) + lane_row
            p = ones
            for w in range(_W - 1, -1, -1):
                ch = plsc.load_gather(chars_v, [cidx0 + w])
                m = ch != 0
                wgt = jnp.where(m, p, 0.0)
                plsc.addupdate_scatter(outbuf, [idx0 + ch], wgt)
                p = p * jnp.where(m, ffv, ones)
            return carry

        lax.fori_loop(0, _GROUPS, fill_body, 0)

        pltpu.sync_copy(
            outbuf,
            out_hbm.at[pl.ds((base_word + chunk * _CHUNK) * _VOCAB, _CHUNK * _VOCAB)],
        )

        if chunk != _NCHUNK - 1:

            def rezero_body(g, carry):
                cidx0 = cbase + g * (_L * _W) + lane_w
                idx0 = g * (_L * _VOCAB) + lane_row
                for w in range(_W):
                    ch = plsc.load_gather(chars_v, [cidx0 + w])
                    plsc.store_scatter(outbuf, [idx0 + ch], zeros)
                return carry

            lax.fori_loop(0, _GROUPS, rezero_body, 0)


@functools.partial(jax.jit, static_argnames=())
def _fofe_sc(ff16, x_flat):
    run = pl.kernel(
        _fofe_body,
        out_type=jax.ShapeDtypeStruct((_N_WORDS * _VOCAB,), jnp.float32),
        mesh=plsc.VectorSubcoreMesh(core_axis_name="c", subcore_axis_name="s"),
        compiler_params=pltpu.CompilerParams(needs_layout_passes=False),
        scratch_types=[
            pltpu.VMEM((_L,), jnp.float32),
            pltpu.VMEM((_WPW * _W,), jnp.int32),
            pltpu.VMEM((_CHUNK * _VOCAB,), jnp.float32),
        ],
    )
    return run(ff16, x_flat)


def kernel(sents, lengths, forgetting_factor):
    B, S, Wd = sents.shape
    x = sents.reshape(B * S * Wd).astype(jnp.int32)
    ff16 = jnp.broadcast_to(forgetting_factor.astype(jnp.float32), (_L,))
    out = _fofe_sc(ff16, x)
    return out.reshape(B, S, _VOCAB), lengths


# speedup vs baseline: 4.5259x; 1.0925x over previous
import functools

import jax
import jax.numpy as jnp
from jax import lax
from jax.experimental import pallas as pl
from jax.experimental.pallas import tpu as pltpu
from jax.experimental.pallas import tpu_sc as plsc

_VOCAB = 128
_W = 16
_L = 16
_NC = 2
_NS = 16
_NWORKERS = _NC * _NS
_B = 16
_S = 2048
_N_WORDS = _B * _S
_WPW = _N_WORDS // _NWORKERS
_CHUNK = 256
_NCHUNK = _WPW // _CHUNK
_GROUPS = _CHUNK // _L


def _fofe_body(ff_hbm, x_hbm, out_hbm, ff_v, chars_v, buf0, buf1, sem0, sem1):
    cid = lax.axis_index("c")
    sid = lax.axis_index("s")
    wid = sid * _NC + cid
    base_word = wid * _WPW
    b = wid // 2
    s0 = (wid % 2) * _WPW

    pltpu.sync_copy(ff_hbm, ff_v)
    pltpu.sync_copy(x_hbm.at[b, pl.ds(s0, _WPW), :], chars_v)

    ffv = ff_v[...]
    ones = jnp.ones((_L,), jnp.float32)
    zeros = jnp.zeros((_L,), jnp.float32)
    lane = lax.iota(jnp.int32, _L)
    lane_row = lane * _VOCAB
    bufs = (buf0, buf1)
    sems = (sem0, sem1)
    wcols = [jnp.full((_L,), w, jnp.int32) for w in range(_W)]

    def zero_body(i, carry):
        for j in range(8):
            buf0[pl.ds(i * 128 + j * 16, 16)] = zeros
            buf1[pl.ds(i * 128 + j * 16, 16)] = zeros
        return carry

    lax.fori_loop(0, _CHUNK * _VOCAB // 128, zero_body, 0)

    def fill_body(chunk, buf):
        def body(g2, carry):
            for h in range(2):
                g = g2 * 2 + h
                rows = chunk * _CHUNK + g * _L + lane
                idx0 = g * (_L * _VOCAB) + lane_row
                p = ones
                for w in range(_W - 1, -1, -1):
                    ch = plsc.load_gather(chars_v, [rows, wcols[w]])
                    m = ch != 0
                    wgt = jnp.where(m, p, 0.0)
                    plsc.addupdate_scatter(buf, [idx0 + ch], wgt)
                    p = p * jnp.where(m, ffv, ones)
            return carry

        lax.fori_loop(0, _GROUPS // 2, body, 0)

    def rezero_body(chunk, buf):
        def body(g2, carry):
            for h in range(2):
                g = g2 * 2 + h
                rows = chunk * _CHUNK + g * _L + lane
                idx0 = g * (_L * _VOCAB) + lane_row
                for w in range(_W):
                    ch = plsc.load_gather(chars_v, [rows, wcols[w]])
                    plsc.store_scatter(buf, [idx0 + ch], zeros)
            return carry

        lax.fori_loop(0, _GROUPS // 2, body, 0)

    copies = [None] * _NCHUNK
    for chunk in range(_NCHUNK):
        k = chunk % 2
        if chunk >= 2:
            copies[chunk - 2].wait()
            rezero_body(chunk - 2, bufs[k])
        fill_body(chunk, bufs[k])
        copies[chunk] = pltpu.make_async_copy(
            bufs[k],
            out_hbm.at[pl.ds((base_word + chunk * _CHUNK) * _VOCAB, _CHUNK * _VOCAB)],
            sems[k],
        )
        copies[chunk].start()
    copies[_NCHUNK - 2].wait()
    copies[_NCHUNK - 1].wait()


@jax.jit
def _fofe_sc(ff16, x):
    run = pl.kernel(
        _fofe_body,
        out_type=jax.ShapeDtypeStruct((_N_WORDS * _VOCAB,), jnp.float32),
        mesh=plsc.VectorSubcoreMesh(core_axis_name="c", subcore_axis_name="s"),
        compiler_params=pltpu.CompilerParams(
            needs_layout_passes=False, use_tc_tiling_on_sc=False
        ),
        scratch_types=[
            pltpu.VMEM((_L,), jnp.float32),
            pltpu.VMEM((_WPW, _W), jnp.int32),
            pltpu.VMEM((_CHUNK * _VOCAB,), jnp.float32),
            pltpu.VMEM((_CHUNK * _VOCAB,), jnp.float32),
            pltpu.SemaphoreType.DMA,
            pltpu.SemaphoreType.DMA,
        ],
    )
    return run(ff16, x)


def kernel(sents, lengths, forgetting_factor):
    B, S, Wd = sents.shape
    x = sents.astype(jnp.int32)
    ff16 = jnp.broadcast_to(forgetting_factor.astype(jnp.float32), (_L,))
    out = _fofe_sc(ff16, x)
    return out.reshape(B, S, _VOCAB), lengths


# speedup vs baseline: 7.1649x; 1.5831x over previous
import functools

import jax
import jax.numpy as jnp
from jax import lax
from jax.experimental import pallas as pl
from jax.experimental.pallas import tpu as pltpu
from jax.experimental.pallas import tpu_sc as plsc

_VOCAB = 128
_W = 16
_L = 16
_NC = 2
_NS = 16
_NWORKERS = _NC * _NS
_B = 16
_S = 2048
_N_WORDS = _B * _S
_WPW = _N_WORDS // _NWORKERS
_CHUNK = 256
_NCHUNK = _WPW // _CHUNK
_GROUPS = _CHUNK // _L


def _fofe_body(ff_hbm, xt_hbm, out_hbm, ff_v, chars_v, buf0, buf1, sem0, sem1):
    cid = lax.axis_index("c")
    sid = lax.axis_index("s")
    wid = sid * _NC + cid
    base_word = wid * _WPW
    b = wid // 2
    s0 = (wid % 2) * _WPW

    pltpu.sync_copy(ff_hbm, ff_v)
    pltpu.sync_copy(xt_hbm.at[b, :, pl.ds(s0, _WPW)], chars_v)

    ffv = ff_v[...]
    ones = jnp.ones((_L,), jnp.float32)
    zeros = jnp.zeros((_L,), jnp.float32)
    lane = lax.iota(jnp.int32, _L)
    lane_row = lane * _VOCAB
    bufs = (buf0, buf1)
    sems = (sem0, sem1)
    wrows = [jnp.full((_L,), w, jnp.int32) for w in range(_W)]

    def zero_body(i, carry):
        for j in range(8):
            buf0[pl.ds(i * 128 + j * 16, 16)] = zeros
            buf1[pl.ds(i * 128 + j * 16, 16)] = zeros
        return carry

    lax.fori_loop(0, _CHUNK * _VOCAB // 128, zero_body, 0)

    def fill_body(chunk, buf):
        def body(g2, carry):
            for h in range(2):
                g = g2 * 2 + h
                scol = chunk * _CHUNK + g * _L + lane
                idx0 = g * (_L * _VOCAB) + lane_row
                p = ones
                for w in range(_W - 1, -1, -1):
                    ch = plsc.load_gather(chars_v, [wrows[w], scol])
                    m = ch != 0
                    wgt = jnp.where(m, p, 0.0)
                    plsc.addupdate_scatter(buf, [idx0 + ch], wgt)
                    p = p * jnp.where(m, ffv, ones)
            return carry

        lax.fori_loop(0, _GROUPS // 2, body, 0)

    def rezero_body(chunk, buf):
        def body(g2, carry):
            for h in range(2):
                g = g2 * 2 + h
                scol = chunk * _CHUNK + g * _L + lane
                idx0 = g * (_L * _VOCAB) + lane_row
                for w in range(_W):
                    ch = plsc.load_gather(chars_v, [wrows[w], scol])
                    plsc.store_scatter(buf, [idx0 + ch], zeros)
            return carry

        lax.fori_loop(0, _GROUPS // 2, body, 0)

    copies = [None] * _NCHUNK
    for chunk in range(_NCHUNK):
        k = chunk % 2
        if chunk >= 2:
            copies[chunk - 2].wait()
            rezero_body(chunk - 2, bufs[k])
        fill_body(chunk, bufs[k])
        copies[chunk] = pltpu.make_async_copy(
            bufs[k],
            out_hbm.at[pl.ds((base_word + chunk * _CHUNK) * _VOCAB, _CHUNK * _VOCAB)],
            sems[k],
        )
        copies[chunk].start()
    copies[_NCHUNK - 2].wait()
    copies[_NCHUNK - 1].wait()


@jax.jit
def _fofe_sc(ff16, xt):
    run = pl.kernel(
        _fofe_body,
        out_type=jax.ShapeDtypeStruct((_N_WORDS * _VOCAB,), jnp.float32),
        mesh=plsc.VectorSubcoreMesh(core_axis_name="c", subcore_axis_name="s"),
        compiler_params=pltpu.CompilerParams(needs_layout_passes=False),
        scratch_types=[
            pltpu.VMEM((_L,), jnp.float32),
            pltpu.VMEM((_W, _WPW), jnp.int32),
            pltpu.VMEM((_CHUNK * _VOCAB,), jnp.float32),
            pltpu.VMEM((_CHUNK * _VOCAB,), jnp.float32),
            pltpu.SemaphoreType.DMA,
            pltpu.SemaphoreType.DMA,
        ],
    )
    return run(ff16, xt)


def kernel(sents, lengths, forgetting_factor):
    B, S, Wd = sents.shape
    xt = jnp.swapaxes(sents.astype(jnp.int32), 1, 2)
    ff16 = jnp.broadcast_to(forgetting_factor.astype(jnp.float32), (_L,))
    out = _fofe_sc(ff16, xt)
    return out.reshape(B, S, _VOCAB), lengths
